# Initial kernel scaffold; baseline (speedup 1.0000x reference)
import jax
import jax.numpy as jnp
from jax.experimental import pallas as pl


def kernel(Z, scalar_representation, idx_m, W1, b1, W2, b2):
    raise NotImplementedError("write your pallas kernel here")


# speedup vs baseline: 1.4562x; 1.4562x over previous
import math

import jax
import jax.numpy as jnp
from jax import lax
from jax.experimental import pallas as pl

_N_ATOMS = 32768
_N_IN = 128
_N_HIDDEN = 64
_N_ELEMENTS = 10
_N_MOLECULES = 512
_BLOCK = 1024
_LOG2 = math.log(2.0)


def _mlp_kernel(x_ref, z_ref, idx_ref, w1_ref, b1_ref, w2_ref, b2_ref,
                out_ref):
    i = pl.program_id(0)

    @pl.when(i == 0)
    def _():
        out_ref[...] = jnp.zeros_like(out_ref)

    x = x_ref[...]
    h_all = jnp.dot(x, w1_ref[...],
                    preferred_element_type=jnp.float32) + b1_ref[...]
    zc = z_ref[0]
    eoh = (zc == lax.broadcasted_iota(
        jnp.int32, (x.shape[0], _N_ELEMENTS), 1)).astype(jnp.float32)

    h = jnp.zeros((x.shape[0], _N_HIDDEN), dtype=jnp.float32)
    for e in range(_N_ELEMENTS):
        h += h_all[:, e * _N_HIDDEN:(e + 1) * _N_HIDDEN] * eoh[:, e:e + 1]

    h = jnp.maximum(h, 0.0) + jnp.log1p(jnp.exp(-jnp.abs(h))) - _LOG2

    y_all = jnp.dot(h, w2_ref[...],
                    preferred_element_type=jnp.float32) + b2_ref[...]
    y_atom = jnp.sum(y_all * eoh, axis=1, keepdims=True)

    mc = idx_ref[0]
    moh = (mc == lax.broadcasted_iota(
        jnp.int32, (x.shape[0], _N_MOLECULES), 1)).astype(jnp.float32)
    contrib = lax.dot_general(
        y_atom, moh, (((0,), (0,)), ((), ())),
        preferred_element_type=jnp.float32)
    out_ref[...] += contrib


@jax.jit
def kernel(Z, scalar_representation, idx_m, W1, b1, W2, b2):
    n_blocks = _N_ATOMS // _BLOCK
    w1f = W1.transpose(1, 0, 2).reshape(_N_IN, _N_ELEMENTS * _N_HIDDEN)
    b1f = b1.reshape(1, _N_ELEMENTS * _N_HIDDEN)
    w2f = W2[:, :, 0].T
    b2f = b2.reshape(1, _N_ELEMENTS)
    z3 = Z.reshape(n_blocks, _BLOCK, 1)
    idx3 = idx_m.reshape(n_blocks, _BLOCK, 1)

    out = pl.pallas_call(
        _mlp_kernel,
        grid=(n_blocks,),
        in_specs=[
            pl.BlockSpec((_BLOCK, _N_IN), lambda i: (i, 0)),
            pl.BlockSpec((1, _BLOCK, 1), lambda i: (i, 0, 0)),
            pl.BlockSpec((1, _BLOCK, 1), lambda i: (i, 0, 0)),
            pl.BlockSpec((_N_IN, _N_ELEMENTS * _N_HIDDEN), lambda i: (0, 0)),
            pl.BlockSpec((1, _N_ELEMENTS * _N_HIDDEN), lambda i: (0, 0)),
            pl.BlockSpec((_N_HIDDEN, _N_ELEMENTS), lambda i: (0, 0)),
            pl.BlockSpec((1, _N_ELEMENTS), lambda i: (0, 0)),
        ],
        out_specs=pl.BlockSpec((1, _N_MOLECULES), lambda i: (0, 0)),
        out_shape=jax.ShapeDtypeStruct((1, _N_MOLECULES), jnp.float32),
    )(scalar_representation, z3, idx3, w1f, b1f, w2f, b2f)
    return out[0]


# speedup vs baseline: 1.6812x; 1.1545x over previous
import math

import jax
import jax.numpy as jnp
from jax import lax
from jax.experimental import pallas as pl

_N_ATOMS = 32768
_N_IN = 128
_N_HIDDEN = 64
_N_ELEMENTS = 10
_N_MOLECULES = 512
_BLOCK = 1024
_LOG2 = math.log(2.0)


def _mlp_kernel(x_ref, z_ref, idx_ref, w1_ref, b1_ref, w2_ref, b2_ref,
                ex_ref, out_ref):
    i = pl.program_id(0)

    @pl.when(i == 0)
    def _():
        out_ref[...] = jnp.zeros_like(out_ref)

    x = x_ref[...].astype(jnp.bfloat16)
    h_all = jnp.dot(x, w1_ref[...],
                    preferred_element_type=jnp.float32) + b1_ref[...]
    zc = z_ref[0]
    eoh = (zc == lax.broadcasted_iota(
        jnp.int32, (x.shape[0], _N_ELEMENTS), 1)).astype(jnp.bfloat16)
    eoh_wide = jnp.dot(eoh, ex_ref[...],
                       preferred_element_type=jnp.float32)
    masked = h_all * eoh_wide
    h = jnp.zeros((x.shape[0], _N_HIDDEN), dtype=jnp.float32)
    for e in range(_N_ELEMENTS):
        h += masked[:, e * _N_HIDDEN:(e + 1) * _N_HIDDEN]

    h = jnp.maximum(h, 0.0) + jnp.log1p(jnp.exp(-jnp.abs(h))) - _LOG2

    y_all = jnp.dot(h, w2_ref[...],
                    preferred_element_type=jnp.float32) + b2_ref[...]
    y_atom = jnp.sum(y_all * eoh.astype(jnp.float32), axis=1,
                     keepdims=True)

    mc = idx_ref[0]
    moh = (mc == lax.broadcasted_iota(
        jnp.int32, (x.shape[0], _N_MOLECULES), 1)).astype(jnp.float32)
    contrib = lax.dot_general(
        y_atom, moh, (((0,), (0,)), ((), ())),
        preferred_element_type=jnp.float32)
    out_ref[...] += contrib


@jax.jit
def kernel(Z, scalar_representation, idx_m, W1, b1, W2, b2):
    n_blocks = _N_ATOMS // _BLOCK
    w1f = W1.transpose(1, 0, 2).reshape(
        _N_IN, _N_ELEMENTS * _N_HIDDEN).astype(jnp.bfloat16)
    b1f = b1.reshape(1, _N_ELEMENTS * _N_HIDDEN)
    w2f = W2[:, :, 0].T
    b2f = b2.reshape(1, _N_ELEMENTS)
    expand = jnp.repeat(jnp.eye(_N_ELEMENTS, dtype=jnp.bfloat16),
                        _N_HIDDEN, axis=1)
    z3 = Z.reshape(n_blocks, _BLOCK, 1)
    idx3 = idx_m.reshape(n_blocks, _BLOCK, 1)

    out = pl.pallas_call(
        _mlp_kernel,
        grid=(n_blocks,),
        in_specs=[
            pl.BlockSpec((_BLOCK, _N_IN), lambda i: (i, 0)),
            pl.BlockSpec((1, _BLOCK, 1), lambda i: (i, 0, 0)),
            pl.BlockSpec((1, _BLOCK, 1), lambda i: (i, 0, 0)),
            pl.BlockSpec((_N_IN, _N_ELEMENTS * _N_HIDDEN), lambda i: (0, 0)),
            pl.BlockSpec((1, _N_ELEMENTS * _N_HIDDEN), lambda i: (0, 0)),
            pl.BlockSpec((_N_HIDDEN, _N_ELEMENTS), lambda i: (0, 0)),
            pl.BlockSpec((1, _N_ELEMENTS), lambda i: (0, 0)),
            pl.BlockSpec((_N_ELEMENTS, _N_ELEMENTS * _N_HIDDEN),
                         lambda i: (0, 0)),
        ],
        out_specs=pl.BlockSpec((1, _N_MOLECULES), lambda i: (0, 0)),
        out_shape=jax.ShapeDtypeStruct((1, _N_MOLECULES), jnp.float32),
    )(scalar_representation, z3, idx3, w1f, b1f, w2f, b2f, expand)
    return out[0]


# speedup vs baseline: 1.7675x; 1.0513x over previous
import math

import jax
import jax.numpy as jnp
from jax import lax
from jax.experimental import pallas as pl

_N_ATOMS = 32768
_N_IN = 128
_N_HIDDEN = 64
_N_ELEMENTS = 10
_N_MOLECULES = 512
_BLOCK = 2048
_LOG2 = math.log(2.0)


def _mlp_kernel(x_ref, z_ref, idx_ref, w1_ref, b1_ref, w2_ref, b2_ref,
                ex_ref, out_ref):
    i = pl.program_id(0)

    @pl.when(i == 0)
    def _():
        out_ref[...] = jnp.zeros_like(out_ref)

    x = x_ref[...].astype(jnp.bfloat16)
    h_all = jnp.dot(x, w1_ref[...],
                    preferred_element_type=jnp.float32) + b1_ref[...]
    zc = z_ref[0]
    eoh = (zc == lax.broadcasted_iota(
        jnp.int32, (x.shape[0], _N_ELEMENTS), 1)).astype(jnp.bfloat16)
    eoh_wide = jnp.dot(eoh, ex_ref[...],
                       preferred_element_type=jnp.float32)
    masked = h_all * eoh_wide
    h = jnp.zeros((x.shape[0], _N_HIDDEN), dtype=jnp.float32)
    for e in range(_N_ELEMENTS):
        h += masked[:, e * _N_HIDDEN:(e + 1) * _N_HIDDEN]

    h = jnp.maximum(h, 0.0) + jnp.log1p(jnp.exp(-jnp.abs(h))) - _LOG2

    y_all = jnp.dot(h, w2_ref[...],
                    preferred_element_type=jnp.float32) + b2_ref[...]
    y_atom = jnp.sum(y_all * eoh.astype(jnp.float32), axis=1,
                     keepdims=True)

    mc = idx_ref[0]
    moh = (mc == lax.broadcasted_iota(
        jnp.int32, (x.shape[0], _N_MOLECULES), 1)).astype(jnp.float32)
    contrib = lax.dot_general(
        y_atom, moh, (((0,), (0,)), ((), ())),
        preferred_element_type=jnp.float32)
    out_ref[...] += contrib


@jax.jit
def kernel(Z, scalar_representation, idx_m, W1, b1, W2, b2):
    n_blocks = _N_ATOMS // _BLOCK
    w1f = W1.transpose(1, 0, 2).reshape(
        _N_IN, _N_ELEMENTS * _N_HIDDEN).astype(jnp.bfloat16)
    b1f = b1.reshape(1, _N_ELEMENTS * _N_HIDDEN)
    w2f = W2[:, :, 0].T
    b2f = b2.reshape(1, _N_ELEMENTS)
    expand = jnp.repeat(jnp.eye(_N_ELEMENTS, dtype=jnp.bfloat16),
                        _N_HIDDEN, axis=1)
    z3 = Z.reshape(n_blocks, _BLOCK, 1)
    idx3 = idx_m.reshape(n_blocks, _BLOCK, 1)

    out = pl.pallas_call(
        _mlp_kernel,
        grid=(n_blocks,),
        in_specs=[
            pl.BlockSpec((_BLOCK, _N_IN), lambda i: (i, 0)),
            pl.BlockSpec((1, _BLOCK, 1), lambda i: (i, 0, 0)),
            pl.BlockSpec((1, _BLOCK, 1), lambda i: (i, 0, 0)),
            pl.BlockSpec((_N_IN, _N_ELEMENTS * _N_HIDDEN), lambda i: (0, 0)),
            pl.BlockSpec((1, _N_ELEMENTS * _N_HIDDEN), lambda i: (0, 0)),
            pl.BlockSpec((_N_HIDDEN, _N_ELEMENTS), lambda i: (0, 0)),
            pl.BlockSpec((1, _N_ELEMENTS), lambda i: (0, 0)),
            pl.BlockSpec((_N_ELEMENTS, _N_ELEMENTS * _N_HIDDEN),
                         lambda i: (0, 0)),
        ],
        out_specs=pl.BlockSpec((1, _N_MOLECULES), lambda i: (0, 0)),
        out_shape=jax.ShapeDtypeStruct((1, _N_MOLECULES), jnp.float32),
    )(scalar_representation, z3, idx3, w1f, b1f, w2f, b2f, expand)
    return out[0]


# speedup vs baseline: 1.8695x; 1.0577x over previous
import math

import jax
import jax.numpy as jnp
from jax import lax
from jax.experimental import pallas as pl

_N_ATOMS = 32768
_N_IN = 128
_N_HIDDEN = 64
_N_ELEMENTS = 10
_N_MOLECULES = 512
_BLOCK = 2048
_LOG2 = math.log(2.0)


def _mlp_kernel(x_ref, z_ref, idx_ref, w1_ref, b1_ref, w2_ref, b2_ref,
                ex_ref, ones_ref, fold_ref, out_ref):
    i = pl.program_id(0)

    @pl.when(i == 0)
    def _():
        out_ref[...] = jnp.zeros_like(out_ref)

    x = x_ref[...].astype(jnp.bfloat16)
    h_all = jnp.dot(x, w1_ref[...],
                    preferred_element_type=jnp.float32)
    zc = z_ref[0]
    eoh = (zc == lax.broadcasted_iota(
        jnp.int32, (x.shape[0], _N_ELEMENTS), 1)).astype(jnp.bfloat16)
    eoh_wide = jnp.dot(eoh, ex_ref[...],
                       preferred_element_type=jnp.float32)
    masked = h_all * eoh_wide
    h = (jnp.dot(eoh, b1_ref[...], preferred_element_type=jnp.float32) +
         jnp.dot(masked, fold_ref[...],
                 preferred_element_type=jnp.float32))

    h = jnp.maximum(h, 0.0) + jnp.log1p(jnp.exp(-jnp.abs(h))) - _LOG2

    w2sel = jnp.dot(eoh, w2_ref[...],
                    preferred_element_type=jnp.float32)
    b2sel = jnp.dot(eoh, b2_ref[...],
                    preferred_element_type=jnp.float32)
    y_atom = jnp.dot(h * w2sel, ones_ref[...],
                     preferred_element_type=jnp.float32) + b2sel

    mc = idx_ref[0]
    moh = (mc == lax.broadcasted_iota(
        jnp.int32, (x.shape[0], _N_MOLECULES), 1)).astype(jnp.float32)
    contrib = lax.dot_general(
        y_atom, moh, (((0,), (0,)), ((), ())),
        preferred_element_type=jnp.float32)
    out_ref[...] += contrib


@jax.jit
def kernel(Z, scalar_representation, idx_m, W1, b1, W2, b2):
    n_blocks = _N_ATOMS // _BLOCK
    w1f = W1.transpose(1, 0, 2).reshape(
        _N_IN, _N_ELEMENTS * _N_HIDDEN).astype(jnp.bfloat16)
    b1f = b1.astype(jnp.bfloat16)
    w2f = W2[:, :, 0].astype(jnp.bfloat16)
    b2f = b2.astype(jnp.bfloat16)
    expand = jnp.repeat(jnp.eye(_N_ELEMENTS, dtype=jnp.bfloat16),
                        _N_HIDDEN, axis=1)
    ones64 = jnp.ones((_N_HIDDEN, 1), jnp.float32)
    fold = jnp.tile(jnp.eye(_N_HIDDEN, dtype=jnp.float32),
                    (_N_ELEMENTS, 1))
    z3 = Z.reshape(n_blocks, _BLOCK, 1)
    idx3 = idx_m.reshape(n_blocks, _BLOCK, 1)

    out = pl.pallas_call(
        _mlp_kernel,
        grid=(n_blocks,),
        in_specs=[
            pl.BlockSpec((_BLOCK, _N_IN), lambda i: (i, 0)),
            pl.BlockSpec((1, _BLOCK, 1), lambda i: (i, 0, 0)),
            pl.BlockSpec((1, _BLOCK, 1), lambda i: (i, 0, 0)),
            pl.BlockSpec((_N_IN, _N_ELEMENTS * _N_HIDDEN), lambda i: (0, 0)),
            pl.BlockSpec((_N_ELEMENTS, _N_HIDDEN), lambda i: (0, 0)),
            pl.BlockSpec((_N_ELEMENTS, _N_HIDDEN), lambda i: (0, 0)),
            pl.BlockSpec((_N_ELEMENTS, 1), lambda i: (0, 0)),
            pl.BlockSpec((_N_ELEMENTS, _N_ELEMENTS * _N_HIDDEN),
                         lambda i: (0, 0)),
            pl.BlockSpec((_N_HIDDEN, 1), lambda i: (0, 0)),
            pl.BlockSpec((_N_ELEMENTS * _N_HIDDEN, _N_HIDDEN),
                         lambda i: (0, 0)),
        ],
        out_specs=pl.BlockSpec((1, _N_MOLECULES), lambda i: (0, 0)),
        out_shape=jax.ShapeDtypeStruct((1, _N_MOLECULES), jnp.float32),
    )(scalar_representation, z3, idx3, w1f, b1f, w2f, b2f, expand, ones64,
      fold)
    return out.reshape(_N_MOLECULES)
